# baseline (speedup 1.0000x reference)
import functools

import jax
import jax.numpy as jnp
from jax import lax
from jax.experimental import pallas as pl
from jax.experimental.pallas import tpu as pltpu
from jax.experimental.pallas import tpu_sc as plsc

B = 16384
BH = B // 2
G = 50
D = 64
C_PAD = 128
T_COL = 100
T_ROWS = 1024
NW = 32
RB = BH // NW
N_GRP = RB // 16
N_CHUNK = 4


def _sc_body(half, t_hbm, genres_hbm, counts_hbm, t_v, gen_v, counts_v, gsem, csem):
    wid = lax.axis_index("s") * 2 + lax.axis_index("c")
    base = wid * RB
    gbase = half * BH + base

    gen_cp = pltpu.async_copy(genres_hbm.at[:, pl.ds(gbase, RB)], gen_v, gsem)
    pltpu.sync_copy(t_hbm.at[pl.ds(gbase, RB)], t_v)

    iota16 = lax.broadcasted_iota(jnp.int32, (16,), 0)
    ones16 = jnp.ones((16,), jnp.float32)
    zeros16 = jnp.zeros((16,), jnp.float32)
    tcol16 = jnp.full((16,), T_COL, jnp.int32)

    def zrow(r4, carry):
        for rr in range(4):
            for c in range(C_PAD // 16):
                counts_v[r4 * 4 + rr, pl.ds(c * 16, 16)] = zeros16
        return carry

    lax.fori_loop(0, RB // 4, zrow, 0)
    gen_cp.wait()

    grp_per_chunk = N_GRP // N_CHUNK
    rows_per_chunk = RB // N_CHUNK

    def grp(i, carry):
        rows16 = i * 16 + iota16
        for s in range(G):
            g16 = gen_v[s, pl.ds(i * 16, 16)]
            plsc.addupdate_scatter(counts_v, [rows16, g16], ones16)
        t16 = t_v[pl.ds(i * 16, 16)].astype(jnp.float32)
        plsc.store_scatter(counts_v, [rows16, tcol16], t16)
        return carry

    ccopies = []
    for ch in range(N_CHUNK):
        lax.fori_loop(ch * grp_per_chunk, (ch + 1) * grp_per_chunk, grp, 0)
        r0 = ch * rows_per_chunk
        ccopies.append(pltpu.async_copy(
            counts_v.at[pl.ds(r0, rows_per_chunk)],
            counts_hbm.at[pl.ds(base + r0, rows_per_chunk)], csem))

    for cp in ccopies:
        cp.wait()


def _tc_body(counts_ref, ttab_ref, stab_ref, out_ref):
    rt = counts_ref.shape[0]
    counts = counts_ref[...]
    tcol = counts[:, T_COL:T_COL + 1].astype(jnp.int32)
    oh_t = (tcol == lax.broadcasted_iota(jnp.int32, (rt, T_ROWS), 1)
            ).astype(jnp.bfloat16)
    tpart = jnp.dot(oh_t, ttab_ref[...], preferred_element_type=jnp.float32)
    styles = jnp.dot(counts.astype(jnp.bfloat16), stab_ref[...],
                     preferred_element_type=jnp.float32)
    out_ref[...] = jnp.concatenate([tpart, styles], axis=1)


def _tc_body2(counts_ref, ttab_ref, stab_ref, prev_ref, out_ref):
    _tc_body(counts_ref, ttab_ref, stab_ref, out_ref)


_SC_PARAMS = dict(
    compiler_params=pltpu.CompilerParams(
        needs_layout_passes=False, use_tc_tiling_on_sc=False),
    out_type=jax.ShapeDtypeStruct((BH, C_PAD), jnp.float32),
    scratch_types=[
        pltpu.VMEM((RB,), jnp.int32),
        pltpu.VMEM((G, RB), jnp.int32),
        pltpu.VMEM((RB, C_PAD), jnp.float32),
        pltpu.SemaphoreType.DMA,
        pltpu.SemaphoreType.DMA,
    ],
)

RT = 2048


@jax.jit
def kernel(t, genres, t_table, style_table):
    t1 = t.astype(jnp.int32)
    genres_t = genres.astype(jnp.int32).T
    ttab = jnp.zeros((T_ROWS, D), jnp.bfloat16).at[: t_table.shape[0]].set(
        t_table.astype(jnp.bfloat16))
    stab = jnp.zeros((C_PAD, D), jnp.bfloat16).at[: style_table.shape[0]].set(
        style_table.astype(jnp.bfloat16))

    mesh = plsc.VectorSubcoreMesh(core_axis_name="c", subcore_axis_name="s")
    counts0 = pl.kernel(functools.partial(_sc_body, 0), mesh=mesh, **_SC_PARAMS)(
        t1, genres_t)
    counts1 = pl.kernel(functools.partial(_sc_body, 1), mesh=mesh, **_SC_PARAMS)(
        t1, genres_t)

    grid_h = BH // RT
    out0 = pl.pallas_call(
        _tc_body,
        grid=(grid_h,),
        in_specs=[
            pl.BlockSpec((RT, C_PAD), lambda i: (i, 0)),
            pl.BlockSpec((T_ROWS, D), lambda i: (0, 0)),
            pl.BlockSpec((C_PAD, D), lambda i: (0, 0)),
        ],
        out_specs=pl.BlockSpec((RT, 128), lambda i: (i, 0)),
        out_shape=jax.ShapeDtypeStruct((B, 128), jnp.float32),
    )(counts0, ttab, stab)
    out = pl.pallas_call(
        _tc_body2,
        grid=(grid_h,),
        in_specs=[
            pl.BlockSpec((RT, C_PAD), lambda i: (i, 0)),
            pl.BlockSpec((T_ROWS, D), lambda i: (0, 0)),
            pl.BlockSpec((C_PAD, D), lambda i: (0, 0)),
            pl.BlockSpec((8, 128), lambda i: (0, 0)),
        ],
        out_specs=pl.BlockSpec((RT, 128), lambda i: (i + grid_h, 0)),
        out_shape=jax.ShapeDtypeStruct((B, 128), jnp.float32),
        input_output_aliases={3: 0},
    )(counts1, ttab, stab, out0)
    return out[:, :, None]


# speedup vs baseline: 1.0428x; 1.0428x over previous
import functools

import jax
import jax.numpy as jnp
from jax import lax
from jax.experimental import pallas as pl
from jax.experimental.pallas import tpu as pltpu
from jax.experimental.pallas import tpu_sc as plsc

B = 16384
B0 = 10240
B1 = B - B0
G = 50
D = 64
C_PAD = 128
T_COL = 100
T_ROWS = 1024
NW = 32
N_CHUNK = 4


def _sc_body(start, nrows, t_hbm, genres_hbm, counts_hbm,
             t_v, gen_v, counts_v, gsem, csem):
    rb = nrows // NW
    n_grp = rb // 16
    wid = lax.axis_index("s") * 2 + lax.axis_index("c")
    base = wid * rb
    gbase = start + base

    gen_cp = pltpu.async_copy(genres_hbm.at[:, pl.ds(gbase, rb)], gen_v, gsem)
    pltpu.sync_copy(t_hbm.at[pl.ds(gbase, rb)], t_v)

    iota16 = lax.broadcasted_iota(jnp.int32, (16,), 0)
    ones16 = jnp.ones((16,), jnp.float32)
    zeros16 = jnp.zeros((16,), jnp.float32)
    tcol16 = jnp.full((16,), T_COL, jnp.int32)

    def zrow(r4, carry):
        for rr in range(4):
            for c in range(C_PAD // 16):
                counts_v[r4 * 4 + rr, pl.ds(c * 16, 16)] = zeros16
        return carry

    lax.fori_loop(0, rb // 4, zrow, 0)
    gen_cp.wait()

    grp_per_chunk = n_grp // N_CHUNK
    rows_per_chunk = rb // N_CHUNK

    def grp(i, carry):
        rows16 = i * 16 + iota16
        for s in range(G):
            g16 = gen_v[s, pl.ds(i * 16, 16)]
            plsc.addupdate_scatter(counts_v, [rows16, g16], ones16)
        t16 = t_v[pl.ds(i * 16, 16)].astype(jnp.float32)
        plsc.store_scatter(counts_v, [rows16, tcol16], t16)
        return carry

    ccopies = []
    for ch in range(N_CHUNK):
        lax.fori_loop(ch * grp_per_chunk, (ch + 1) * grp_per_chunk, grp, 0)
        r0 = ch * rows_per_chunk
        ccopies.append(pltpu.async_copy(
            counts_v.at[pl.ds(r0, rows_per_chunk)],
            counts_hbm.at[pl.ds(base + r0, rows_per_chunk)], csem))

    for cp in ccopies:
        cp.wait()


def _tc_body(counts_ref, ttab_ref, stab_ref, out_ref):
    rt = counts_ref.shape[0]
    counts = counts_ref[...]
    tcol = counts[:, T_COL:T_COL + 1].astype(jnp.int32)
    oh_t = (tcol == lax.broadcasted_iota(jnp.int32, (rt, T_ROWS), 1)
            ).astype(jnp.bfloat16)
    tpart = jnp.dot(oh_t, ttab_ref[...], preferred_element_type=jnp.float32)
    styles = jnp.dot(counts.astype(jnp.bfloat16), stab_ref[...],
                     preferred_element_type=jnp.float32)
    out_ref[...] = jnp.concatenate([tpart, styles], axis=1)


def _tc_body2(counts_ref, ttab_ref, stab_ref, prev_ref, out_ref):
    _tc_body(counts_ref, ttab_ref, stab_ref, out_ref)


def _sc_params(nrows):
    rb = nrows // NW
    return dict(
        compiler_params=pltpu.CompilerParams(
            needs_layout_passes=False, use_tc_tiling_on_sc=False),
        out_type=jax.ShapeDtypeStruct((nrows, C_PAD), jnp.float32),
        scratch_types=[
            pltpu.VMEM((rb,), jnp.int32),
            pltpu.VMEM((G, rb), jnp.int32),
            pltpu.VMEM((rb, C_PAD), jnp.float32),
            pltpu.SemaphoreType.DMA,
            pltpu.SemaphoreType.DMA,
        ],
    )


RT = 2048


@jax.jit
def kernel(t, genres, t_table, style_table):
    t1 = t.astype(jnp.int32)
    genres_t = genres.astype(jnp.int32).T
    ttab = jnp.zeros((T_ROWS, D), jnp.bfloat16).at[: t_table.shape[0]].set(
        t_table.astype(jnp.bfloat16))
    stab = jnp.zeros((C_PAD, D), jnp.bfloat16).at[: style_table.shape[0]].set(
        style_table.astype(jnp.bfloat16))

    mesh = plsc.VectorSubcoreMesh(core_axis_name="c", subcore_axis_name="s")
    counts0 = pl.kernel(functools.partial(_sc_body, 0, B0), mesh=mesh,
                        **_sc_params(B0))(t1, genres_t)
    counts1 = pl.kernel(functools.partial(_sc_body, B0, B1), mesh=mesh,
                        **_sc_params(B1))(t1, genres_t)

    g0 = B0 // RT
    g1 = B1 // RT
    out0 = pl.pallas_call(
        _tc_body,
        grid=(g0,),
        in_specs=[
            pl.BlockSpec((RT, C_PAD), lambda i: (i, 0)),
            pl.BlockSpec((T_ROWS, D), lambda i: (0, 0)),
            pl.BlockSpec((C_PAD, D), lambda i: (0, 0)),
        ],
        out_specs=pl.BlockSpec((RT, 128), lambda i: (i, 0)),
        out_shape=jax.ShapeDtypeStruct((B, 128), jnp.float32),
    )(counts0, ttab, stab)
    out = pl.pallas_call(
        _tc_body2,
        grid=(g1,),
        in_specs=[
            pl.BlockSpec((RT, C_PAD), lambda i: (i, 0)),
            pl.BlockSpec((T_ROWS, D), lambda i: (0, 0)),
            pl.BlockSpec((C_PAD, D), lambda i: (0, 0)),
            pl.BlockSpec((8, 128), lambda i: (0, 0)),
        ],
        out_specs=pl.BlockSpec((RT, 128), lambda i, g=g0: (i + g, 0)),
        out_shape=jax.ShapeDtypeStruct((B, 128), jnp.float32),
        input_output_aliases={3: 0},
    )(counts1, ttab, stab, out0)
    return out[:, :, None]


# speedup vs baseline: 1.0787x; 1.0344x over previous
import functools

import jax
import jax.numpy as jnp
from jax import lax
from jax.experimental import pallas as pl
from jax.experimental.pallas import tpu as pltpu
from jax.experimental.pallas import tpu_sc as plsc

B = 16384
B0 = 12288
B1 = B - B0
G = 50
D = 64
C_PAD = 128
T_COL = 100
T_ROWS = 1024
NW = 32
N_CHUNK = 4


def _sc_body(start, nrows, t_hbm, genres_hbm, counts_hbm,
             t_v, gen_v, counts_v, gsem, csem):
    rb = nrows // NW
    n_grp = rb // 16
    wid = lax.axis_index("s") * 2 + lax.axis_index("c")
    base = wid * rb
    gbase = start + base

    gen_cp = pltpu.async_copy(genres_hbm.at[:, pl.ds(gbase, rb)], gen_v, gsem)
    pltpu.sync_copy(t_hbm.at[pl.ds(gbase, rb)], t_v)

    iota16 = lax.broadcasted_iota(jnp.int32, (16,), 0)
    ones16 = jnp.ones((16,), jnp.float32)
    zeros16 = jnp.zeros((16,), jnp.float32)
    tcol16 = jnp.full((16,), T_COL, jnp.int32)

    def zrow(r4, carry):
        for rr in range(4):
            for c in range(C_PAD // 16):
                counts_v[r4 * 4 + rr, pl.ds(c * 16, 16)] = zeros16
        return carry

    lax.fori_loop(0, rb // 4, zrow, 0)
    gen_cp.wait()

    grp_per_chunk = n_grp // N_CHUNK
    rows_per_chunk = rb // N_CHUNK

    def grp(i, carry):
        rows16 = i * 16 + iota16
        for s in range(G):
            g16 = gen_v[s, pl.ds(i * 16, 16)]
            plsc.addupdate_scatter(counts_v, [rows16, g16], ones16)
        t16 = t_v[pl.ds(i * 16, 16)].astype(jnp.float32)
        plsc.store_scatter(counts_v, [rows16, tcol16], t16)
        return carry

    ccopies = []
    for ch in range(N_CHUNK):
        lax.fori_loop(ch * grp_per_chunk, (ch + 1) * grp_per_chunk, grp, 0)
        r0 = ch * rows_per_chunk
        ccopies.append(pltpu.async_copy(
            counts_v.at[pl.ds(r0, rows_per_chunk)],
            counts_hbm.at[pl.ds(base + r0, rows_per_chunk)], csem))

    for cp in ccopies:
        cp.wait()


def _tc_body(counts_ref, ttab_ref, stab_ref, out_ref):
    rt = counts_ref.shape[0]
    counts = counts_ref[...]
    tcol = counts[:, T_COL:T_COL + 1].astype(jnp.int32)
    oh_t = (tcol == lax.broadcasted_iota(jnp.int32, (rt, T_ROWS), 1)
            ).astype(jnp.bfloat16)
    tpart = jnp.dot(oh_t, ttab_ref[...], preferred_element_type=jnp.float32)
    styles = jnp.dot(counts.astype(jnp.bfloat16), stab_ref[...],
                     preferred_element_type=jnp.float32)
    out_ref[...] = jnp.concatenate([tpart, styles], axis=1)


def _tc_body2(counts_ref, ttab_ref, stab_ref, prev_ref, out_ref):
    _tc_body(counts_ref, ttab_ref, stab_ref, out_ref)


def _sc_params(nrows):
    rb = nrows // NW
    return dict(
        compiler_params=pltpu.CompilerParams(
            needs_layout_passes=False, use_tc_tiling_on_sc=True),
        out_type=jax.ShapeDtypeStruct((nrows, C_PAD), jnp.float32),
        scratch_types=[
            pltpu.VMEM((rb,), jnp.int32),
            pltpu.VMEM((G, rb), jnp.int32),
            pltpu.VMEM((rb, C_PAD), jnp.float32),
            pltpu.SemaphoreType.DMA,
            pltpu.SemaphoreType.DMA,
        ],
    )


RT = 2048


@jax.jit
def kernel(t, genres, t_table, style_table):
    t1 = t.astype(jnp.int32)
    genres_t = genres.astype(jnp.int32).T
    ttab = jnp.zeros((T_ROWS, D), jnp.bfloat16).at[: t_table.shape[0]].set(
        t_table.astype(jnp.bfloat16))
    stab = jnp.zeros((C_PAD, D), jnp.bfloat16).at[: style_table.shape[0]].set(
        style_table.astype(jnp.bfloat16))

    mesh = plsc.VectorSubcoreMesh(core_axis_name="c", subcore_axis_name="s")
    counts0 = pl.kernel(functools.partial(_sc_body, 0, B0), mesh=mesh,
                        **_sc_params(B0))(t1, genres_t)
    counts1 = pl.kernel(functools.partial(_sc_body, B0, B1), mesh=mesh,
                        **_sc_params(B1))(t1, genres_t)

    g0 = B0 // RT
    g1 = B1 // RT
    out0 = pl.pallas_call(
        _tc_body,
        grid=(g0,),
        in_specs=[
            pl.BlockSpec((RT, C_PAD), lambda i: (i, 0)),
            pl.BlockSpec((T_ROWS, D), lambda i: (0, 0)),
            pl.BlockSpec((C_PAD, D), lambda i: (0, 0)),
        ],
        out_specs=pl.BlockSpec((RT, 128), lambda i: (i, 0)),
        out_shape=jax.ShapeDtypeStruct((B, 128), jnp.float32),
    )(counts0, ttab, stab)
    out = pl.pallas_call(
        _tc_body2,
        grid=(g1,),
        in_specs=[
            pl.BlockSpec((RT, C_PAD), lambda i: (i, 0)),
            pl.BlockSpec((T_ROWS, D), lambda i: (0, 0)),
            pl.BlockSpec((C_PAD, D), lambda i: (0, 0)),
            pl.BlockSpec((8, 128), lambda i: (0, 0)),
        ],
        out_specs=pl.BlockSpec((RT, 128), lambda i, g=g0: (i + g, 0)),
        out_shape=jax.ShapeDtypeStruct((B, 128), jnp.float32),
        input_output_aliases={3: 0},
    )(counts1, ttab, stab, out0)
    return out[:, :, None]
